# baseline (speedup 1.0000x reference)
import functools

import jax
import jax.numpy as jnp
from jax import lax
from jax.experimental import pallas as pl
from jax.experimental.pallas import tpu as pltpu
from jax.experimental.pallas import tpu_sc as plsc

NUM_OBJ_CLS = 151
NUM_REL_CLS = 51
NUM_OBJS = 4096
NUM_RELS = 65536
DPAD = 64

NC = 2
NS = 16
L = 16
NW = NC * NS
B_PER_W = NUM_RELS // NW
CHUNK = 128
N_CHUNKS = B_PER_W // CHUNK


def _onehot_body(labels_ref, out_ref):
    lbl = labels_ref[...]
    iot = lax.broadcasted_iota(jnp.int32, (NUM_OBJS, NUM_OBJ_CLS), 1)
    out_ref[...] = (lbl == iot).astype(jnp.float32)


_onehot = pl.pallas_call(
    _onehot_body,
    out_shape=jax.ShapeDtypeStruct((NUM_OBJS, NUM_OBJ_CLS), jnp.float32),
)


def _rel_body(labels_hbm, pairs_hbm, table_hbm, out_hbm,
              labels_v, pairs_v, idx_v, rows_a, rows_b, sem_g, sem_wa, sem_wb):
    wid = lax.axis_index("s") * NC + lax.axis_index("c")
    base = wid * B_PER_W

    pltpu.sync_copy(labels_hbm, labels_v)
    pltpu.sync_copy(pairs_hbm.at[pl.ds(base, B_PER_W)], pairs_v)

    lane = lax.broadcasted_iota(jnp.int32, (L,), 0)
    col0 = jnp.zeros((L,), jnp.int32)
    col1 = jnp.ones((L,), jnp.int32)

    def step(i, carry):
        rows = i * L + lane
        h = plsc.load_gather(pairs_v, [rows, col0])
        t = plsc.load_gather(pairs_v, [rows, col1])
        hl = plsc.load_gather(labels_v, [h])
        tl = plsc.load_gather(labels_v, [t])
        idx_v[pl.ds(i * L, L)] = hl * NUM_OBJ_CLS + tl
        return carry

    lax.fori_loop(0, B_PER_W // L, step, 0)

    rows = [rows_a, rows_b]
    sems = [sem_wa, sem_wb]
    wb = [None, None]
    for k in range(N_CHUNKS):
        b = k % 2
        if wb[b] is not None:
            wb[b].wait()
        pltpu.async_copy(
            table_hbm.at[idx_v.at[pl.ds(k * CHUNK, CHUNK)]],
            rows[b], sem_g).wait()
        wb[b] = pltpu.async_copy(
            rows[b], out_hbm.at[pl.ds(base + k * CHUNK, CHUNK)], sems[b])
    wb[0].wait()
    wb[1].wait()


_rel_gather = pl.kernel(
    _rel_body,
    out_type=jax.ShapeDtypeStruct((NUM_RELS, DPAD), jnp.float32),
    mesh=plsc.VectorSubcoreMesh(
        core_axis_name="c", subcore_axis_name="s",
        num_cores=NC, num_subcores=NS),
    scratch_types=[
        pltpu.VMEM((NUM_OBJS,), jnp.int32),
        pltpu.VMEM((B_PER_W, 2), jnp.int32),
        pltpu.VMEM((B_PER_W,), jnp.int32),
        pltpu.VMEM((CHUNK, DPAD), jnp.float32),
        pltpu.VMEM((CHUNK, DPAD), jnp.float32),
        pltpu.SemaphoreType.DMA,
        pltpu.SemaphoreType.DMA,
        pltpu.SemaphoreType.DMA,
    ],
    compiler_params=pltpu.CompilerParams(
        needs_layout_passes=False, use_tc_tiling_on_sc=False),
)


@jax.jit
def kernel(obj_labels, rel_pair_idxs, prior_table):
    labels = obj_labels.astype(jnp.int32)
    pairs = rel_pair_idxs.astype(jnp.int32)
    table2d = prior_table.reshape(NUM_OBJ_CLS * NUM_OBJ_CLS, NUM_REL_CLS)
    table64 = jnp.pad(table2d, ((0, 0), (0, DPAD - NUM_REL_CLS)))
    obj_dists = _onehot(labels.reshape(NUM_OBJS, 1))
    rel64 = _rel_gather(labels, pairs, table64)
    return (obj_dists, rel64[:, :NUM_REL_CLS])


# speedup vs baseline: 1.1658x; 1.1658x over previous
import functools

import jax
import jax.numpy as jnp
from jax import lax
from jax.experimental import pallas as pl
from jax.experimental.pallas import tpu as pltpu
from jax.experimental.pallas import tpu_sc as plsc

NUM_OBJ_CLS = 151
NUM_REL_CLS = 51
NUM_OBJS = 4096
NUM_RELS = 65536
DPAD = 64

NC = 2
NS = 16
L = 16
NW = NC * NS
B_PER_W = NUM_RELS // NW
CHUNK = 128
N_CHUNKS = B_PER_W // CHUNK
OH_PER_W = NUM_OBJS // NW
OH_WORDS = OH_PER_W * NUM_OBJ_CLS


def _idx_oh_body(labels_hbm, pairs_hbm, oh_hbm, idx_hbm,
                 labels_v, pairs_v, idx_v, oh_v):
    wid = lax.axis_index("s") * NC + lax.axis_index("c")
    base = wid * B_PER_W

    pltpu.sync_copy(labels_hbm, labels_v)
    pltpu.sync_copy(pairs_hbm.at[pl.ds(2 * base, 2 * B_PER_W)], pairs_v)

    lane = lax.broadcasted_iota(jnp.int32, (L,), 0)
    lane2 = 2 * lane
    zero16 = jnp.zeros((L,), jnp.float32)
    one16 = jnp.ones((L,), jnp.float32)

    def zstep(i, carry):
        oh_v[pl.ds(i * L, L)] = zero16
        return carry
    lax.fori_loop(0, OH_WORDS // L, zstep, 0)

    obase = wid * OH_PER_W
    for j in range(OH_PER_W // L):
        r16 = j * L + lane
        lbl = plsc.load_gather(labels_v, [obase + r16])
        plsc.store_scatter(oh_v, [r16 * NUM_OBJ_CLS + lbl], one16)
    pltpu.sync_copy(oh_v, oh_hbm.at[pl.ds(wid * OH_WORDS, OH_WORDS)])

    def step(i, carry):
        pos = i * (2 * L) + lane2
        h = plsc.load_gather(pairs_v, [pos])
        t = plsc.load_gather(pairs_v, [pos + 1])
        hl = plsc.load_gather(labels_v, [h])
        tl = plsc.load_gather(labels_v, [t])
        idx_v[pl.ds(i * L, L)] = hl * NUM_OBJ_CLS + tl
        return carry
    lax.fori_loop(0, B_PER_W // L, step, 0)
    pltpu.sync_copy(idx_v, idx_hbm.at[pl.ds(base, B_PER_W)])


_idx_oh = pl.kernel(
    _idx_oh_body,
    out_type=(
        jax.ShapeDtypeStruct((NUM_OBJS * NUM_OBJ_CLS,), jnp.float32),
        jax.ShapeDtypeStruct((NUM_RELS,), jnp.int32),
    ),
    mesh=plsc.VectorSubcoreMesh(
        core_axis_name="c", subcore_axis_name="s",
        num_cores=NC, num_subcores=NS),
    scratch_types=[
        pltpu.VMEM((NUM_OBJS,), jnp.int32),
        pltpu.VMEM((2 * B_PER_W,), jnp.int32),
        pltpu.VMEM((B_PER_W,), jnp.int32),
        pltpu.VMEM((OH_WORDS,), jnp.float32),
    ],
    compiler_params=pltpu.CompilerParams(
        needs_layout_passes=False, use_tc_tiling_on_sc=False),
)


def _rel_body(idx_hbm, table_hbm, out_hbm,
              idx_v, rows_a, rows_b, sem_g, sem_wa, sem_wb):
    wid = lax.axis_index("s") * NC + lax.axis_index("c")
    base = wid * B_PER_W

    pltpu.sync_copy(idx_hbm.at[pl.ds(base, B_PER_W)], idx_v)

    rows = [rows_a, rows_b]
    sems = [sem_wa, sem_wb]
    wb = [None, None]
    for k in range(N_CHUNKS):
        b = k % 2
        if wb[b] is not None:
            wb[b].wait()
        pltpu.async_copy(
            table_hbm.at[idx_v.at[pl.ds(k * CHUNK, CHUNK)]],
            rows[b], sem_g).wait()
        wb[b] = pltpu.async_copy(
            rows[b], out_hbm.at[pl.ds(base + k * CHUNK, CHUNK)], sems[b])
    wb[0].wait()
    wb[1].wait()


_rel_gather = pl.kernel(
    _rel_body,
    out_type=jax.ShapeDtypeStruct((NUM_RELS, DPAD), jnp.float32),
    mesh=plsc.VectorSubcoreMesh(
        core_axis_name="c", subcore_axis_name="s",
        num_cores=NC, num_subcores=NS),
    scratch_types=[
        pltpu.VMEM((B_PER_W,), jnp.int32),
        pltpu.VMEM((CHUNK, DPAD), jnp.float32),
        pltpu.VMEM((CHUNK, DPAD), jnp.float32),
        pltpu.SemaphoreType.DMA,
        pltpu.SemaphoreType.DMA,
        pltpu.SemaphoreType.DMA,
    ],
    compiler_params=pltpu.CompilerParams(
        needs_layout_passes=False, use_tc_tiling_on_sc=False),
)


@jax.jit
def kernel(obj_labels, rel_pair_idxs, prior_table):
    labels = obj_labels.astype(jnp.int32)
    pairs = rel_pair_idxs.astype(jnp.int32).reshape(2 * NUM_RELS)
    table64 = lax.pad(
        prior_table, jnp.float32(0.0),
        ((0, 0, 0), (0, 0, 0), (0, DPAD - NUM_REL_CLS, 0)),
    ).reshape(NUM_OBJ_CLS * NUM_OBJ_CLS, DPAD)
    oh_flat, idx = _idx_oh(labels, pairs)
    rel64 = _rel_gather(idx, table64)
    return (oh_flat.reshape(NUM_OBJS, NUM_OBJ_CLS), rel64[:, :NUM_REL_CLS])
